# baseline (speedup 1.0000x reference)
import functools

import jax
import jax.numpy as jnp
from jax import lax
from jax.experimental import pallas as pl
from jax.experimental.pallas import tpu as pltpu
from jax.experimental.pallas import tpu_sc as plsc

NUM_CORES = 2
NUM_SUBCORES = 16
LANES = 16
NUM_WORKERS = NUM_CORES * NUM_SUBCORES

BATCH = 16384
DIM = 32
CHUNK = BATCH // NUM_WORKERS
NGROUPS = CHUNK // LANES


def _body(user_hbm, item_hbm, ue_hbm, ie_hbm, ub_hbm, ib_hbm, off_hbm,
          out_hbm, uidx_v, iidx_v, blk_v, uc_v, bias_v, out_v, off_v, sem):
    wid = lax.axis_index("s") * NUM_CORES + lax.axis_index("c")
    base = wid * CHUNK

    pltpu.sync_copy(user_hbm.at[pl.ds(base, CHUNK)], uidx_v)
    pltpu.sync_copy(item_hbm.at[pl.ds(base, CHUNK)], iidx_v)
    pltpu.sync_copy(off_hbm, off_v.at[pl.ds(0, 1)])
    off = off_v[pl.ds(0, LANES)][0]

    ivec = lax.iota(jnp.int32, LANES)

    def fetch_blocks(table_hbm, r):
        copies = []
        for i in range(LANES):
            start = pl.multiple_of((r[i] // 128) * 128, 128)
            copies.append(pltpu.async_copy(
                table_hbm.at[:, pl.ds(start, 128)], blk_v.at[i], sem))
        return copies

    def fetch_bias(bias_hbm, idx_ref, g, half):
        return [pltpu.async_copy(
            bias_hbm.at[idx_ref.at[pl.ds(g * LANES, LANES)]],
            bias_v.at[pl.ds(half * LANES, LANES)], sem)]

    def group(g, carry):
        gbase = g * LANES
        ru = uidx_v[pl.ds(gbase, LANES)]
        ri = iidx_v[pl.ds(gbase, LANES)]
        lane_u = ru % 128
        lane_i = ri % 128

        copies = fetch_blocks(ue_hbm, ru)
        copies += fetch_bias(ub_hbm, uidx_v, g, 0)
        copies += fetch_bias(ib_hbm, iidx_v, g, 1)
        for c in copies:
            c.wait()
        for d in range(DIM):
            dvec = jnp.full((LANES,), d, jnp.int32)
            uc_v[d, pl.ds(0, LANES)] = plsc.load_gather(
                blk_v, (ivec, dvec, lane_u))
        ub = bias_v[pl.ds(0, LANES)]
        ib = bias_v[pl.ds(LANES, LANES)]

        copies = fetch_blocks(ie_hbm, ri)
        for c in copies:
            c.wait()
        acc = ub + ib + off
        for d in range(DIM):
            dvec = jnp.full((LANES,), d, jnp.int32)
            acc = acc + uc_v[d, pl.ds(0, LANES)] * plsc.load_gather(
                blk_v, (ivec, dvec, lane_i))

        out_v[pl.ds(gbase, LANES)] = 5.5 / (1.0 + jnp.exp(-acc))
        return carry

    lax.fori_loop(0, NGROUPS, group, 0)
    pltpu.sync_copy(out_v, out_hbm.at[pl.ds(base, CHUNK)])


@jax.jit
def kernel(user, item, user_emb, item_emb, user_bias, item_bias, offset):
    run = functools.partial(
        pl.kernel,
        out_type=jax.ShapeDtypeStruct((BATCH,), jnp.float32),
        mesh=plsc.VectorSubcoreMesh(core_axis_name="c", subcore_axis_name="s"),
        scratch_types=[
            pltpu.VMEM((CHUNK,), jnp.int32),
            pltpu.VMEM((CHUNK,), jnp.int32),
            pltpu.VMEM((LANES, DIM, 128), jnp.float32),
            pltpu.VMEM((DIM, LANES), jnp.float32),
            pltpu.VMEM((2 * LANES,), jnp.float32),
            pltpu.VMEM((CHUNK,), jnp.float32),
            pltpu.VMEM((LANES,), jnp.float32),
            pltpu.SemaphoreType.DMA,
        ],
        compiler_params=pltpu.CompilerParams(
            needs_layout_passes=False, use_tc_tiling_on_sc=True),
    )(_body)
    return run(user.astype(jnp.int32), item.astype(jnp.int32),
               user_emb.T, item_emb.T, user_bias, item_bias, offset)
